# Initial kernel scaffold; baseline (speedup 1.0000x reference)
import jax
import jax.numpy as jnp
from jax.experimental import pallas as pl


def kernel(x, adj, alphas_raw, lin_W, lin_b, fc_W, fc_b):
    raise NotImplementedError("write your pallas kernel here")


# speedup vs baseline: 1.0573x; 1.0573x over previous
import jax
import jax.numpy as jnp
from jax.experimental import pallas as pl
from jax.experimental.pallas import tpu as pltpu

_N = 10000
_F = 128
_DEPTH = 3
_A = 1.0
_B = 1.0
_LB = -1.0
_RB = 1.0


def _pass1_kernel(coef_ref, adj_ref, x0b_ref, x0_ref, x1_ref, adjb_ref):
    adjb = adj_ref[...].astype(jnp.bfloat16)
    adjb_ref[...] = adjb
    y = jnp.dot(adjb, x0b_ref[...], preferred_element_type=jnp.float32)
    x1_ref[...] = coef_ref[0] * y + coef_ref[1] * x0_ref[...]


def _pass1(coefs, adj, x0b, x0, bm):
    nb = _N // bm
    return pl.pallas_call(
        _pass1_kernel,
        grid=(nb,),
        in_specs=[
            pl.BlockSpec(memory_space=pltpu.SMEM),
            pl.BlockSpec((bm, _N), lambda i: (i, 0)),
            pl.BlockSpec((_N, _F), lambda i: (0, 0)),
            pl.BlockSpec((bm, _F), lambda i: (i, 0)),
        ],
        out_specs=[
            pl.BlockSpec((bm, _F), lambda i: (i, 0)),
            pl.BlockSpec((bm, _N), lambda i: (i, 0)),
        ],
        out_shape=[
            jax.ShapeDtypeStruct((_N, _F), jnp.float32),
            jax.ShapeDtypeStruct((_N, _N), jnp.bfloat16),
        ],
    )(coefs, adj, x0b, x0)


def _passL_kernel(coef_ref, adjb_ref, xb_ref, xp_ref, xp2_ref, out_ref):
    y = jnp.dot(adjb_ref[...], xb_ref[...], preferred_element_type=jnp.float32)
    out_ref[...] = (coef_ref[0] * y + coef_ref[1] * xp_ref[...]
                    + coef_ref[2] * xp2_ref[...])


def _passL(coefs, adjb, xprevb, xprev, xprev2, bm):
    nb = _N // bm
    return pl.pallas_call(
        _passL_kernel,
        grid=(nb,),
        in_specs=[
            pl.BlockSpec(memory_space=pltpu.SMEM),
            pl.BlockSpec((bm, _N), lambda i: (i, 0)),
            pl.BlockSpec((_N, _F), lambda i: (0, 0)),
            pl.BlockSpec((bm, _F), lambda i: (i, 0)),
            pl.BlockSpec((bm, _F), lambda i: (i, 0)),
        ],
        out_specs=pl.BlockSpec((bm, _F), lambda i: (i, 0)),
        out_shape=jax.ShapeDtypeStruct((_N, _F), jnp.float32),
    )(coefs, adjb, xprevb, xprev, xprev2)


def _colmean_kernel(x0_ref, x1_ref, x2_ref, x3_ref, out_ref):
    i = pl.program_id(0)

    parts = jnp.concatenate(
        [jnp.sum(ref[...], axis=0, keepdims=True)
         for ref in (x0_ref, x1_ref, x2_ref, x3_ref)],
        axis=0) * (1.0 / _N)

    @pl.when(i == 0)
    def _init():
        out_ref[...] = parts

    @pl.when(i != 0)
    def _acc():
        out_ref[...] = out_ref[...] + parts


def _colmeans(xs, bm):
    nb = _N // bm
    return pl.pallas_call(
        _colmean_kernel,
        grid=(nb,),
        in_specs=[pl.BlockSpec((bm, _F), lambda i: (i, 0)) for _ in range(4)],
        out_specs=pl.BlockSpec((_DEPTH + 1, _F), lambda i: (0, 0)),
        out_shape=jax.ShapeDtypeStruct((_DEPTH + 1, _F), jnp.float32),
    )(*xs)


def _b16(v):
    return v.astype(jnp.bfloat16).astype(jnp.float32)


def _fuse_kernel(q_ref, lwb_ref, lb_ref, fwb_ref, fb_ref,
                 x0_ref, x1_ref, x2_ref, x3_ref, out_ref):
    xprojs = []
    logits = []
    for k, ref in enumerate((x0_ref, x1_ref, x2_ref, x3_ref)):
        xp = jax.lax.dot_general(
            ref[...].astype(jnp.bfloat16), lwb_ref[k],
            (((1,), (1,)), ((), ())),
            preferred_element_type=jnp.float32,
        ) + lb_ref[k][None, :]
        xpb = _b16(xp)
        qb = _b16(q_ref[k])
        t = jnp.tanh(jnp.sum(qb[None, :] * xpb, axis=1, keepdims=True))
        xprojs.append((xp, xpb))
        logits.append(t)
    m = jnp.maximum(jnp.maximum(logits[0], logits[1]),
                    jnp.maximum(logits[2], logits[3]))
    es = [jnp.exp(t - m) for t in logits]
    denom = es[0] + es[1] + es[2] + es[3]
    acc = None
    for k in range(4):
        wb = _b16(es[k] / denom)
        term = wb * xprojs[k][1]
        acc = term if acc is None else acc + term
    out_ref[...] = jax.lax.dot_general(
        acc.astype(jnp.bfloat16), fwb_ref[...],
        (((1,), (1,)), ((), ())),
        preferred_element_type=jnp.float32,
    ) + fb_ref[...]


def _fuse(q, lin_Wb, lin_b, fc_Wb, fc_b, xs, bm):
    nb = _N // bm
    return pl.pallas_call(
        _fuse_kernel,
        grid=(nb,),
        in_specs=[
            pl.BlockSpec((_DEPTH + 1, _F), lambda i: (0, 0)),
            pl.BlockSpec((_DEPTH + 1, _F, _F), lambda i: (0, 0, 0)),
            pl.BlockSpec((_DEPTH + 1, _F), lambda i: (0, 0)),
            pl.BlockSpec((_F, _F), lambda i: (0, 0)),
            pl.BlockSpec((1, _F), lambda i: (0, 0)),
        ] + [pl.BlockSpec((bm, _F), lambda i: (i, 0)) for _ in range(4)],
        out_specs=pl.BlockSpec((bm, _F), lambda i: (i, 0)),
        out_shape=jax.ShapeDtypeStruct((_N, _F), jnp.float32),
    )(q, lin_Wb, lin_b, fc_Wb, fc_b.reshape(1, _F), *xs)


@jax.jit
def kernel(x, adj, alphas_raw, lin_W, lin_b, fc_W, fc_b):
    alphas = jnp.tanh(alphas_raw)
    a, b, l, r = _A, _B, _LB, _RB

    c1 = ((a - b) / 2 - (a + b + 2) / 2 * (l + r) / (r - l)) * alphas[0]
    c2 = ((a + b + 2) / (r - l)) * alphas[0]

    def rec_coefs(L):
        coef_l = 2 * L * (L + a + b) * (2 * L - 2 + a + b)
        coef_lm1_1 = (2 * L + a + b - 1) * (2 * L + a + b) * (2 * L + a + b - 2)
        coef_lm1_2 = (2 * L + a + b - 1) * (a ** 2 - b ** 2)
        coef_lm2 = 2 * (L - 1 + a) * (L - 1 + b) * (2 * L + a + b)
        tmp1 = alphas[L - 1] * (coef_lm1_1 / coef_l)
        tmp2 = alphas[L - 1] * (coef_lm1_2 / coef_l)
        tmp3 = alphas[L - 1] * alphas[L - 2] * (coef_lm2 / coef_l)
        tmp1_2 = tmp1 * (2 / (r - l))
        tmp2_2 = tmp1 * ((r + l) / (r - l)) + tmp2
        return tmp1_2, -tmp2_2, -tmp3

    x0 = x
    x1, adjb = _pass1(jnp.stack([c2, c1]), adj, x0.astype(jnp.bfloat16),
                      x0, 80)
    ca, cb, cc = rec_coefs(2)
    x2 = _passL(jnp.stack([ca, cb, cc]), adjb, x1.astype(jnp.bfloat16),
                x1, x0, 400)
    ca, cb, cc = rec_coefs(3)
    x3 = _passL(jnp.stack([ca, cb, cc]), adjb, x2.astype(jnp.bfloat16),
                x2, x1, 400)

    xs = (x0, x1, x2, x3)
    q = _colmeans(xs, 1000)
    return _fuse(q, lin_W.astype(jnp.bfloat16), lin_b,
                 fc_W.astype(jnp.bfloat16), fc_b, xs, 1000)


# speedup vs baseline: 1.1609x; 1.0979x over previous
import jax
import jax.numpy as jnp
from jax.experimental import pallas as pl
from jax.experimental.pallas import tpu as pltpu

_N = 10000
_F = 128
_DEPTH = 3
_A = 1.0
_B = 1.0
_LB = -1.0
_RB = 1.0


def _pass1_kernel(coef_ref, adj_ref, x0b_ref, x0_ref,
                  x1_ref, x1b_ref, adjb_ref, cs_ref):
    adjb = adj_ref[...].astype(jnp.bfloat16)
    adjb_ref[...] = adjb
    y = jnp.dot(adjb, x0b_ref[...], preferred_element_type=jnp.float32)
    x1 = coef_ref[0] * y + coef_ref[1] * x0_ref[...]
    x1_ref[...] = x1
    x1b_ref[...] = x1.astype(jnp.bfloat16)
    part = jnp.concatenate(
        [jnp.sum(x0_ref[...], axis=0, keepdims=True),
         jnp.sum(x1, axis=0, keepdims=True)], axis=0)

    @pl.when(pl.program_id(0) == 0)
    def _init():
        cs_ref[...] = part

    @pl.when(pl.program_id(0) != 0)
    def _acc():
        cs_ref[...] = cs_ref[...] + part


def _pass1(coefs, adj, x0b, x0, bm):
    nb = _N // bm
    return pl.pallas_call(
        _pass1_kernel,
        grid=(nb,),
        in_specs=[
            pl.BlockSpec(memory_space=pltpu.SMEM),
            pl.BlockSpec((bm, _N), lambda i: (i, 0)),
            pl.BlockSpec((_N, _F), lambda i: (0, 0)),
            pl.BlockSpec((bm, _F), lambda i: (i, 0)),
        ],
        out_specs=[
            pl.BlockSpec((bm, _F), lambda i: (i, 0)),
            pl.BlockSpec((bm, _F), lambda i: (i, 0)),
            pl.BlockSpec((bm, _N), lambda i: (i, 0)),
            pl.BlockSpec((2, _F), lambda i: (0, 0)),
        ],
        out_shape=[
            jax.ShapeDtypeStruct((_N, _F), jnp.float32),
            jax.ShapeDtypeStruct((_N, _F), jnp.bfloat16),
            jax.ShapeDtypeStruct((_N, _N), jnp.bfloat16),
            jax.ShapeDtypeStruct((2, _F), jnp.float32),
        ],
    )(coefs, adj, x0b, x0)


def _passL_kernel(coef_ref, adjb_ref, xb_ref, xp_ref, xp2_ref,
                  out_ref, outb_ref, cs_ref):
    y = jnp.dot(adjb_ref[...], xb_ref[...], preferred_element_type=jnp.float32)
    out = (coef_ref[0] * y + coef_ref[1] * xp_ref[...]
           + coef_ref[2] * xp2_ref[...])
    out_ref[...] = out
    outb_ref[...] = out.astype(jnp.bfloat16)
    part = jnp.sum(out, axis=0, keepdims=True)

    @pl.when(pl.program_id(0) == 0)
    def _init():
        cs_ref[...] = part

    @pl.when(pl.program_id(0) != 0)
    def _acc():
        cs_ref[...] = cs_ref[...] + part


def _passL(coefs, adjb, xprevb, xprev, xprev2, bm):
    nb = _N // bm
    return pl.pallas_call(
        _passL_kernel,
        grid=(nb,),
        in_specs=[
            pl.BlockSpec(memory_space=pltpu.SMEM),
            pl.BlockSpec((bm, _N), lambda i: (i, 0)),
            pl.BlockSpec((_N, _F), lambda i: (0, 0)),
            pl.BlockSpec((bm, _F), lambda i: (i, 0)),
            pl.BlockSpec((bm, _F), lambda i: (i, 0)),
        ],
        out_specs=[
            pl.BlockSpec((bm, _F), lambda i: (i, 0)),
            pl.BlockSpec((bm, _F), lambda i: (i, 0)),
            pl.BlockSpec((1, _F), lambda i: (0, 0)),
        ],
        out_shape=[
            jax.ShapeDtypeStruct((_N, _F), jnp.float32),
            jax.ShapeDtypeStruct((_N, _F), jnp.bfloat16),
            jax.ShapeDtypeStruct((1, _F), jnp.float32),
        ],
    )(coefs, adjb, xprevb, xprev, xprev2)


def _b16(v):
    return v.astype(jnp.bfloat16).astype(jnp.float32)


def _fuse_kernel(q_ref, lwb_ref, lb_ref, fwb_ref, fb_ref,
                 x0_ref, x1_ref, x2_ref, x3_ref, out_ref):
    xprojs = []
    logits = []
    for k, ref in enumerate((x0_ref, x1_ref, x2_ref, x3_ref)):
        xp = jax.lax.dot_general(
            ref[...].astype(jnp.bfloat16), lwb_ref[k],
            (((1,), (1,)), ((), ())),
            preferred_element_type=jnp.float32,
        ) + lb_ref[k][None, :]
        xpb = _b16(xp)
        qb = _b16(q_ref[k])
        t = jnp.tanh(jnp.sum(qb[None, :] * xpb, axis=1, keepdims=True))
        xprojs.append(xpb)
        logits.append(t)
    m = jnp.maximum(jnp.maximum(logits[0], logits[1]),
                    jnp.maximum(logits[2], logits[3]))
    es = [jnp.exp(t - m) for t in logits]
    denom = es[0] + es[1] + es[2] + es[3]
    acc = None
    for k in range(4):
        wb = _b16(es[k] / denom)
        term = wb * xprojs[k]
        acc = term if acc is None else acc + term
    out_ref[...] = jax.lax.dot_general(
        acc.astype(jnp.bfloat16), fwb_ref[...],
        (((1,), (1,)), ((), ())),
        preferred_element_type=jnp.float32,
    ) + fb_ref[...]


def _fuse(q, lin_Wb, lin_b, fc_Wb, fc_b, xs, bm):
    nb = _N // bm
    return pl.pallas_call(
        _fuse_kernel,
        grid=(nb,),
        in_specs=[
            pl.BlockSpec((_DEPTH + 1, _F), lambda i: (0, 0)),
            pl.BlockSpec((_DEPTH + 1, _F, _F), lambda i: (0, 0, 0)),
            pl.BlockSpec((_DEPTH + 1, _F), lambda i: (0, 0)),
            pl.BlockSpec((_F, _F), lambda i: (0, 0)),
            pl.BlockSpec((1, _F), lambda i: (0, 0)),
        ] + [pl.BlockSpec((bm, _F), lambda i: (i, 0)) for _ in range(4)],
        out_specs=pl.BlockSpec((bm, _F), lambda i: (i, 0)),
        out_shape=jax.ShapeDtypeStruct((_N, _F), jnp.float32),
    )(q, lin_Wb, lin_b, fc_Wb, fc_b.reshape(1, _F), *xs)


@jax.jit
def kernel(x, adj, alphas_raw, lin_W, lin_b, fc_W, fc_b):
    alphas = jnp.tanh(alphas_raw)
    a, b, l, r = _A, _B, _LB, _RB

    c1 = ((a - b) / 2 - (a + b + 2) / 2 * (l + r) / (r - l)) * alphas[0]
    c2 = ((a + b + 2) / (r - l)) * alphas[0]

    def rec_coefs(L):
        coef_l = 2 * L * (L + a + b) * (2 * L - 2 + a + b)
        coef_lm1_1 = (2 * L + a + b - 1) * (2 * L + a + b) * (2 * L + a + b - 2)
        coef_lm1_2 = (2 * L + a + b - 1) * (a ** 2 - b ** 2)
        coef_lm2 = 2 * (L - 1 + a) * (L - 1 + b) * (2 * L + a + b)
        tmp1 = alphas[L - 1] * (coef_lm1_1 / coef_l)
        tmp2 = alphas[L - 1] * (coef_lm1_2 / coef_l)
        tmp3 = alphas[L - 1] * alphas[L - 2] * (coef_lm2 / coef_l)
        tmp1_2 = tmp1 * (2 / (r - l))
        tmp2_2 = tmp1 * ((r + l) / (r - l)) + tmp2
        return tmp1_2, -tmp2_2, -tmp3

    x0 = x
    x1, x1b, adjb, cs01 = _pass1(jnp.stack([c2, c1]), adj,
                                 x0.astype(jnp.bfloat16), x0, 400)
    ca, cb, cc = rec_coefs(2)
    x2, x2b, cs2 = _passL(jnp.stack([ca, cb, cc]), adjb, x1b, x1, x0, 400)
    ca, cb, cc = rec_coefs(3)
    x3, _, cs3 = _passL(jnp.stack([ca, cb, cc]), adjb, x2b, x2, x1, 400)

    q = jnp.concatenate([cs01, cs2, cs3], axis=0) * (1.0 / _N)
    xs = (x0, x1, x2, x3)
    return _fuse(q, lin_W.astype(jnp.bfloat16), lin_b,
                 fc_W.astype(jnp.bfloat16), fc_b, xs, 1000)


# speedup vs baseline: 1.1972x; 1.0313x over previous
import jax
import jax.numpy as jnp
from jax.experimental import pallas as pl
from jax.experimental.pallas import tpu as pltpu

_N = 10000
_F = 128
_DEPTH = 3
_A = 1.0
_B = 1.0
_LB = -1.0
_RB = 1.0


def _pass1_kernel(coef_ref, adj_ref, x0b_ref, x0_ref,
                  x1_ref, x1b_ref, adjb_ref, cs_ref):
    adjb = adj_ref[...].astype(jnp.bfloat16)
    adjb_ref[...] = adjb
    y = jnp.dot(adjb, x0b_ref[...], preferred_element_type=jnp.float32)
    x1 = coef_ref[0] * y + coef_ref[1] * x0_ref[...]
    x1_ref[...] = x1
    x1b_ref[...] = x1.astype(jnp.bfloat16)
    part = jnp.concatenate(
        [jnp.sum(x0_ref[...], axis=0, keepdims=True),
         jnp.sum(x1, axis=0, keepdims=True)], axis=0)

    @pl.when(pl.program_id(0) == 0)
    def _init():
        cs_ref[...] = part

    @pl.when(pl.program_id(0) != 0)
    def _acc():
        cs_ref[...] = cs_ref[...] + part


def _pass1(coefs, adj, x0b, x0, bm):
    nb = _N // bm
    return pl.pallas_call(
        _pass1_kernel,
        grid=(nb,),
        in_specs=[
            pl.BlockSpec(memory_space=pltpu.SMEM),
            pl.BlockSpec((bm, _N), lambda i: (i, 0)),
            pl.BlockSpec((_N, _F), lambda i: (0, 0)),
            pl.BlockSpec((bm, _F), lambda i: (i, 0)),
        ],
        out_specs=[
            pl.BlockSpec((bm, _F), lambda i: (i, 0)),
            pl.BlockSpec((bm, _F), lambda i: (i, 0)),
            pl.BlockSpec((bm, _N), lambda i: (i, 0)),
            pl.BlockSpec((2, _F), lambda i: (0, 0)),
        ],
        out_shape=[
            jax.ShapeDtypeStruct((_N, _F), jnp.float32),
            jax.ShapeDtypeStruct((_N, _F), jnp.bfloat16),
            jax.ShapeDtypeStruct((_N, _N), jnp.bfloat16),
            jax.ShapeDtypeStruct((2, _F), jnp.float32),
        ],
    )(coefs, adj, x0b, x0)


def _passL_kernel(coef_ref, adjb_ref, xb_ref, xp_ref, xp2_ref,
                  *out_refs):
    if len(out_refs) == 3:
        out_ref, outb_ref, cs_ref = out_refs
    else:
        out_ref, (outb_ref, cs_ref) = None, out_refs
    y = jnp.dot(adjb_ref[...], xb_ref[...], preferred_element_type=jnp.float32)
    out = (coef_ref[0] * y + coef_ref[1] * xp_ref[...]
           + coef_ref[2] * xp2_ref[...])
    if out_ref is not None:
        out_ref[...] = out
    outb_ref[...] = out.astype(jnp.bfloat16)
    part = jnp.sum(out, axis=0, keepdims=True)

    @pl.when(pl.program_id(0) == 0)
    def _init():
        cs_ref[...] = part

    @pl.when(pl.program_id(0) != 0)
    def _acc():
        cs_ref[...] = cs_ref[...] + part


def _passL(coefs, adjb, xprevb, xprev, xprev2, bm, want_f32=True):
    nb = _N // bm
    out_specs = [
        pl.BlockSpec((bm, _F), lambda i: (i, 0)),
        pl.BlockSpec((1, _F), lambda i: (0, 0)),
    ]
    out_shape = [
        jax.ShapeDtypeStruct((_N, _F), jnp.bfloat16),
        jax.ShapeDtypeStruct((1, _F), jnp.float32),
    ]
    if want_f32:
        out_specs.insert(0, pl.BlockSpec((bm, _F), lambda i: (i, 0)))
        out_shape.insert(0, jax.ShapeDtypeStruct((_N, _F), jnp.float32))
    return pl.pallas_call(
        _passL_kernel,
        grid=(nb,),
        in_specs=[
            pl.BlockSpec(memory_space=pltpu.SMEM),
            pl.BlockSpec((bm, _N), lambda i: (i, 0)),
            pl.BlockSpec((_N, _F), lambda i: (0, 0)),
            pl.BlockSpec((bm, _F), lambda i: (i, 0)),
            pl.BlockSpec((bm, _F), lambda i: (i, 0)),
        ],
        out_specs=out_specs,
        out_shape=out_shape,
    )(coefs, adjb, xprevb, xprev, xprev2)


def _b16(v):
    return v.astype(jnp.bfloat16).astype(jnp.float32)


def _fuse_kernel(q_ref, lwb_ref, lb_ref, fwb_ref, fb_ref,
                 x0_ref, x1_ref, x2_ref, x3_ref, out_ref):
    xprojs = []
    logits = []
    for k, ref in enumerate((x0_ref, x1_ref, x2_ref, x3_ref)):
        xp = jax.lax.dot_general(
            ref[...], lwb_ref[k],
            (((1,), (1,)), ((), ())),
            preferred_element_type=jnp.float32,
        ) + lb_ref[k][None, :]
        xpb = _b16(xp)
        qb = _b16(q_ref[k])
        t = jnp.tanh(jnp.sum(qb[None, :] * xpb, axis=1, keepdims=True))
        xprojs.append(xpb)
        logits.append(t)
    m = jnp.maximum(jnp.maximum(logits[0], logits[1]),
                    jnp.maximum(logits[2], logits[3]))
    es = [jnp.exp(t - m) for t in logits]
    denom = es[0] + es[1] + es[2] + es[3]
    acc = None
    for k in range(4):
        wb = _b16(es[k] / denom)
        term = wb * xprojs[k]
        acc = term if acc is None else acc + term
    out_ref[...] = jax.lax.dot_general(
        acc.astype(jnp.bfloat16), fwb_ref[...],
        (((1,), (1,)), ((), ())),
        preferred_element_type=jnp.float32,
    ) + fb_ref[...]


def _fuse(q, lin_Wb, lin_b, fc_Wb, fc_b, xs, bm):
    nb = _N // bm
    return pl.pallas_call(
        _fuse_kernel,
        grid=(nb,),
        in_specs=[
            pl.BlockSpec((_DEPTH + 1, _F), lambda i: (0, 0)),
            pl.BlockSpec((_DEPTH + 1, _F, _F), lambda i: (0, 0, 0)),
            pl.BlockSpec((_DEPTH + 1, _F), lambda i: (0, 0)),
            pl.BlockSpec((_F, _F), lambda i: (0, 0)),
            pl.BlockSpec((1, _F), lambda i: (0, 0)),
        ] + [pl.BlockSpec((bm, _F), lambda i: (i, 0)) for _ in range(4)],
        out_specs=pl.BlockSpec((bm, _F), lambda i: (i, 0)),
        out_shape=jax.ShapeDtypeStruct((_N, _F), jnp.float32),
    )(q, lin_Wb, lin_b, fc_Wb, fc_b.reshape(1, _F), *xs)


@jax.jit
def kernel(x, adj, alphas_raw, lin_W, lin_b, fc_W, fc_b):
    alphas = jnp.tanh(alphas_raw)
    a, b, l, r = _A, _B, _LB, _RB

    c1 = ((a - b) / 2 - (a + b + 2) / 2 * (l + r) / (r - l)) * alphas[0]
    c2 = ((a + b + 2) / (r - l)) * alphas[0]

    def rec_coefs(L):
        coef_l = 2 * L * (L + a + b) * (2 * L - 2 + a + b)
        coef_lm1_1 = (2 * L + a + b - 1) * (2 * L + a + b) * (2 * L + a + b - 2)
        coef_lm1_2 = (2 * L + a + b - 1) * (a ** 2 - b ** 2)
        coef_lm2 = 2 * (L - 1 + a) * (L - 1 + b) * (2 * L + a + b)
        tmp1 = alphas[L - 1] * (coef_lm1_1 / coef_l)
        tmp2 = alphas[L - 1] * (coef_lm1_2 / coef_l)
        tmp3 = alphas[L - 1] * alphas[L - 2] * (coef_lm2 / coef_l)
        tmp1_2 = tmp1 * (2 / (r - l))
        tmp2_2 = tmp1 * ((r + l) / (r - l)) + tmp2
        return tmp1_2, -tmp2_2, -tmp3

    x0 = x
    x0b = x0.astype(jnp.bfloat16)
    x1, x1b, adjb, cs01 = _pass1(jnp.stack([c2, c1]), adj, x0b, x0, 400)
    ca, cb, cc = rec_coefs(2)
    x2, x2b, cs2 = _passL(jnp.stack([ca, cb, cc]), adjb, x1b, x1, x0, 1000)
    ca, cb, cc = rec_coefs(3)
    x3b, cs3 = _passL(jnp.stack([ca, cb, cc]), adjb, x2b, x2, x1, 1000,
                      want_f32=False)

    q = jnp.concatenate([cs01, cs2, cs3], axis=0) * (1.0 / _N)
    xsb = (x0b, x1b, x2b, x3b)
    return _fuse(q, lin_W.astype(jnp.bfloat16), lin_b,
                 fc_W.astype(jnp.bfloat16), fc_b, xsb, 2000)
